# baseline (speedup 1.0000x reference)
import jax
import jax.numpy as jnp
from jax import lax
from jax.experimental import pallas as pl
from jax.experimental.pallas import tpu as pltpu
from jax.experimental.pallas import tpu_sc as plsc

NC = 2
NS = 16
LANES = 16


def _mesh():
    return plsc.VectorSubcoreMesh(
        core_axis_name="c", subcore_axis_name="s",
        num_cores=NC, num_subcores=NS)


def _fill(ref, n, value):
    v = jnp.full((LANES,), value, jnp.float32)

    def body(k, carry):
        ref[pl.ds(k * LANES, LANES)] = v
        return carry
    lax.fori_loop(0, n // LANES, body, 0)


def _edge_pass1(n_pad, e, chunk, lam0, lam1, y0, y1, send, recv, w):
    node_chunk = n_pad // NS
    per_w = e // (NC * NS)
    n_chunks = per_w // chunk
    assert n_chunks % 2 == 0 and n_chunks >= 4

    def body(lam0_h, lam1_h, y0_h, y1_h, send_h, recv_h, w_h,
             o_la0, o_la1, o_ya0, o_ya1, o_wd, o_w2, o_dg,
             sh_la0, sh_la1, sh_ya0, sh_ya1, sh_wd, sh_w2, sh_dg,
             zb, one_v,
             s_v0, r_v0, w_v0, g0_0, g1_0, g2_0, g3_0, rs_0, ws_0, w2_0,
             s_v1, r_v1, w_v1, g0_1, g1_1, g2_1, g3_1, rs_1, ws_1, w2_1,
             semL0, semL1, semG0, semG1, semS0, semS1):
        c = lax.axis_index("c")
        s = lax.axis_index("s")
        wid = c * NS + s
        nodelo = s * node_chunk

        tables = (sh_la0, sh_la1, sh_ya0, sh_ya1, sh_wd, sh_w2, sh_dg)
        outs = (o_la0, o_la1, o_ya0, o_ya1, o_wd, o_w2, o_dg)
        gsrc = (lam0_h, lam1_h, y0_h, y1_h)
        sets = (
            dict(s_v=s_v0, r_v=r_v0, w_v=w_v0, g=(g0_0, g1_0, g2_0, g3_0),
                 rs=rs_0, ws=ws_0, w2=w2_0, semL=semL0, semG=semG0,
                 semS=semS0),
            dict(s_v=s_v1, r_v=r_v1, w_v=w_v1, g=(g0_1, g1_1, g2_1, g3_1),
                 rs=rs_1, ws=ws_1, w2=w2_1, semL=semL1, semG=semG1,
                 semS=semS1),
        )

        _fill(zb, node_chunk, 0.0)
        for t in tables:
            pltpu.sync_copy(zb, t.at[pl.ds(nodelo, node_chunk)])
        _fill(one_v, chunk, 1.0)
        plsc.subcore_barrier()

        ebase = wid * per_w

        def lin_start(j, st):
            base = ebase + j * chunk
            pltpu.async_copy(send_h.at[pl.ds(base, chunk)], st["s_v"], st["semL"])
            pltpu.async_copy(recv_h.at[pl.ds(base, chunk)], st["r_v"], st["semL"])
            pltpu.async_copy(w_h.at[pl.ds(base, chunk)], st["w_v"], st["semL"])

        def lin_wait(st):
            pltpu.make_async_copy(send_h.at[pl.ds(ebase, chunk)], st["s_v"], st["semL"]).wait()
            pltpu.make_async_copy(recv_h.at[pl.ds(ebase, chunk)], st["r_v"], st["semL"]).wait()
            pltpu.make_async_copy(w_h.at[pl.ds(ebase, chunk)], st["w_v"], st["semL"]).wait()

        def gath_start(st):
            for src, dst in zip(gsrc, st["g"]):
                pltpu.async_copy(src.at[st["s_v"]], dst, st["semG"])

        def gath_wait(st):
            for src, dst in zip(gsrc, st["g"]):
                pltpu.make_async_copy(src.at[st["s_v"]], dst, st["semG"]).wait()

        def compute(st):
            w_v, r_v = st["w_v"], st["r_v"]
            g0, g1, g2, g3 = st["g"]
            rs, ws, w2 = st["rs"], st["ws"], st["w2"]

            def mul_body(k, carry2):
                sl = pl.ds(k * LANES, LANES)
                wv = w_v[sl]
                nw = -wv
                g0[sl] = nw * g0[sl]
                g1[sl] = nw * g1[sl]
                g2[sl] = nw * g2[sl]
                g3[sl] = nw * g3[sl]
                ws[sl] = wv
                w2[sl] = wv * wv
                rs[sl] = r_v[sl]
                return carry2
            lax.fori_loop(0, chunk // LANES, mul_body, 0)

        def scat_srcs(st):
            return st["g"] + (st["ws"], st["w2"], one_v)

        def scat_start(st):
            rs = st["rs"]
            for src, t in zip(scat_srcs(st), tables):
                pltpu.async_copy(src, t.at[rs], st["semS"], add=True)

        def scat_wait(st):
            rs = st["rs"]
            for src, t in zip(scat_srcs(st), tables):
                pltpu.make_async_copy(src, t.at[rs], st["semS"]).wait()

        def stage(j, p, first, do_n1, do_n2):
            cur, nxt = sets[p], sets[1 - p]
            gath_wait(cur)
            compute(cur)
            scat_start(cur)
            if do_n2:
                lin_start(j + 2, cur)
            if do_n1:
                lin_wait(nxt)
                if not first:
                    scat_wait(nxt)
                gath_start(nxt)

        lin_start(0, sets[0])
        lin_wait(sets[0])
        gath_start(sets[0])
        lin_start(1, sets[1])

        stage(0, 0, True, True, True)
        stage(1, 1, False, True, True)

        def pair_body(j2, carry):
            j = 2 + j2 * 2
            stage(j, 0, False, True, True)
            stage(j + 1, 1, False, True, True)
            return carry
        lax.fori_loop(0, (n_chunks - 4) // 2, pair_body, 0)
        stage(n_chunks - 2, 0, False, True, False)
        stage(n_chunks - 1, 1, False, False, False)
        scat_wait(sets[0])
        scat_wait(sets[1])

        plsc.subcore_barrier()
        for t, o in zip(tables, outs):
            pltpu.sync_copy(t.at[pl.ds(nodelo, node_chunk)],
                            o.at[c, pl.ds(nodelo, node_chunk)])

    part = jax.ShapeDtypeStruct((NC, n_pad), jnp.float32)
    ebuf = ([pltpu.VMEM((chunk,), jnp.int32)] * 2
            + [pltpu.VMEM((chunk,), jnp.float32)] * 5
            + [pltpu.VMEM((chunk,), jnp.int32)]
            + [pltpu.VMEM((chunk,), jnp.float32)] * 2)
    fn = pl.kernel(
        body,
        out_type=(part,) * 7,
        mesh=_mesh(),
        scratch_types=(
            [pltpu.VMEM_SHARED((n_pad,), jnp.float32)] * 7
            + [pltpu.VMEM((node_chunk,), jnp.float32),
               pltpu.VMEM((chunk,), jnp.float32)]
            + ebuf + ebuf
            + [pltpu.SemaphoreType.DMA] * 6),
    )
    return fn(lam0, lam1, y0, y1, send, recv, w)


def _edge_pass2(n_pad, e, chunk, nx0, nx1, send, recv, w):
    node_chunk = n_pad // NS
    per_w = e // (NC * NS)
    n_chunks = per_w // chunk
    assert n_chunks % 2 == 0 and n_chunks >= 4

    def body(nx0_h, nx1_h, send_h, recv_h, w_h,
             o_xa0, o_xa1,
             sh_xa0, sh_xa1,
             zb,
             s_v0, r_v0, w_v0, g0_0, g1_0, rs_0,
             s_v1, r_v1, w_v1, g0_1, g1_1, rs_1,
             semL0, semL1, semG0, semG1, semS0, semS1):
        c = lax.axis_index("c")
        s = lax.axis_index("s")
        wid = c * NS + s
        nodelo = s * node_chunk

        tables = (sh_xa0, sh_xa1)
        gsrc = (nx0_h, nx1_h)
        sets = (
            dict(s_v=s_v0, r_v=r_v0, w_v=w_v0, g=(g0_0, g1_0),
                 rs=rs_0, semL=semL0, semG=semG0, semS=semS0),
            dict(s_v=s_v1, r_v=r_v1, w_v=w_v1, g=(g0_1, g1_1),
                 rs=rs_1, semL=semL1, semG=semG1, semS=semS1),
        )

        _fill(zb, node_chunk, 0.0)
        pltpu.sync_copy(zb, sh_xa0.at[pl.ds(nodelo, node_chunk)])
        pltpu.sync_copy(zb, sh_xa1.at[pl.ds(nodelo, node_chunk)])
        plsc.subcore_barrier()

        ebase = wid * per_w

        def lin_start(j, st):
            base = ebase + j * chunk
            pltpu.async_copy(send_h.at[pl.ds(base, chunk)], st["s_v"], st["semL"])
            pltpu.async_copy(recv_h.at[pl.ds(base, chunk)], st["r_v"], st["semL"])
            pltpu.async_copy(w_h.at[pl.ds(base, chunk)], st["w_v"], st["semL"])

        def lin_wait(st):
            pltpu.make_async_copy(send_h.at[pl.ds(ebase, chunk)], st["s_v"], st["semL"]).wait()
            pltpu.make_async_copy(recv_h.at[pl.ds(ebase, chunk)], st["r_v"], st["semL"]).wait()
            pltpu.make_async_copy(w_h.at[pl.ds(ebase, chunk)], st["w_v"], st["semL"]).wait()

        def gath_start(st):
            for src, dst in zip(gsrc, st["g"]):
                pltpu.async_copy(src.at[st["s_v"]], dst, st["semG"])

        def gath_wait(st):
            for src, dst in zip(gsrc, st["g"]):
                pltpu.make_async_copy(src.at[st["s_v"]], dst, st["semG"]).wait()

        def compute(st):
            w_v, r_v = st["w_v"], st["r_v"]
            g0, g1 = st["g"]
            rs = st["rs"]

            def mul_body(k, carry2):
                sl = pl.ds(k * LANES, LANES)
                nw = -w_v[sl]
                g0[sl] = nw * g0[sl]
                g1[sl] = nw * g1[sl]
                rs[sl] = r_v[sl]
                return carry2
            lax.fori_loop(0, chunk // LANES, mul_body, 0)

        def scat_start(st):
            for src, t in zip(st["g"], tables):
                pltpu.async_copy(src, t.at[st["rs"]], st["semS"], add=True)

        def scat_wait(st):
            for src, t in zip(st["g"], tables):
                pltpu.make_async_copy(src, t.at[st["rs"]], st["semS"]).wait()

        def stage(j, p, first, do_n1, do_n2):
            cur, nxt = sets[p], sets[1 - p]
            gath_wait(cur)
            compute(cur)
            scat_start(cur)
            if do_n2:
                lin_start(j + 2, cur)
            if do_n1:
                lin_wait(nxt)
                if not first:
                    scat_wait(nxt)
                gath_start(nxt)

        lin_start(0, sets[0])
        lin_wait(sets[0])
        gath_start(sets[0])
        lin_start(1, sets[1])

        stage(0, 0, True, True, True)
        stage(1, 1, False, True, True)

        def pair_body(j2, carry):
            j = 2 + j2 * 2
            stage(j, 0, False, True, True)
            stage(j + 1, 1, False, True, True)
            return carry
        lax.fori_loop(0, (n_chunks - 4) // 2, pair_body, 0)
        stage(n_chunks - 2, 0, False, True, False)
        stage(n_chunks - 1, 1, False, False, False)
        scat_wait(sets[0])
        scat_wait(sets[1])

        plsc.subcore_barrier()
        pltpu.sync_copy(sh_xa0.at[pl.ds(nodelo, node_chunk)],
                        o_xa0.at[c, pl.ds(nodelo, node_chunk)])
        pltpu.sync_copy(sh_xa1.at[pl.ds(nodelo, node_chunk)],
                        o_xa1.at[c, pl.ds(nodelo, node_chunk)])

    part = jax.ShapeDtypeStruct((NC, n_pad), jnp.float32)
    ebuf = ([pltpu.VMEM((chunk,), jnp.int32)] * 2
            + [pltpu.VMEM((chunk,), jnp.float32)] * 3
            + [pltpu.VMEM((chunk,), jnp.int32)])
    fn = pl.kernel(
        body,
        out_type=(part, part),
        mesh=_mesh(),
        scratch_types=(
            [pltpu.VMEM_SHARED((n_pad,), jnp.float32)] * 2
            + [pltpu.VMEM((node_chunk,), jnp.float32)]
            + ebuf + ebuf
            + [pltpu.SemaphoreType.DMA] * 6),
    )
    return fn(nx0, nx1, send, recv, w)


def _node_x(x0, x1, y0, y1, lam0, lam1, b0, b1,
            p_la0, p_la1, p_ya0, p_ya1, p_wd, p_w2, p_dg):
    def body(x0_r, x1_r, y0_r, y1_r, lam0_r, lam1_r, b0_r, b1_r,
             la0_r, la1_r, ya0_r, ya1_r, wd_r, w2_r, dg_r,
             nx0_o, nx1_o, wd_o, invd_o):
        la0 = la0_r[0] + la0_r[1]
        la1 = la1_r[0] + la1_r[1]
        ya0 = ya0_r[0] + ya0_r[1]
        ya1 = ya1_r[0] + ya1_r[1]
        wd = wd_r[0] + wd_r[1]
        dsq = w2_r[0] + w2_r[1]
        dg = dg_r[0] + dg_r[1]
        mii = wd * wd + dsq
        den = 1.0 / (2.0 + mii)
        nx0_o[...] = (2.0 * b0_r[...] - la0 - wd * lam0_r[...]
                      + mii * x0_r[...] - ya0 - wd * y0_r[...]) * den
        nx1_o[...] = (2.0 * b1_r[...] - la1 - wd * lam1_r[...]
                      + mii * x1_r[...] - ya1 - wd * y1_r[...]) * den
        wd_o[...] = wd
        invd_o[...] = 1.0 / (1.0 + dg)

    shp = x0.shape
    out = jax.ShapeDtypeStruct(shp, jnp.float32)
    return pl.pallas_call(
        body, out_shape=(out,) * 4,
    )(x0, x1, y0, y1, lam0, lam1, b0, b1,
      p_la0, p_la1, p_ya0, p_ya1, p_wd, p_w2, p_dg)


def _node_ylam(p_xa0, p_xa1, nx0, nx1, wd, invd, lam0, lam1):
    def body(xa0_r, xa1_r, nx0_r, nx1_r, wd_r, invd_r, lam0_r, lam1_r,
             ny0_o, ny1_o, nl0_o, nl1_o):
        xa0 = xa0_r[0] + xa0_r[1]
        xa1 = xa1_r[0] + xa1_r[1]
        ny0 = invd_r[...] * (xa0 + wd_r[...] * nx0_r[...])
        ny1 = invd_r[...] * (xa1 + wd_r[...] * nx1_r[...])
        ny0_o[...] = ny0
        ny1_o[...] = ny1
        nl0_o[...] = lam0_r[...] + ny0
        nl1_o[...] = lam1_r[...] + ny1

    shp = nx0.shape
    out = jax.ShapeDtypeStruct(shp, jnp.float32)
    return pl.pallas_call(
        body, out_shape=(out,) * 4,
    )(p_xa0, p_xa1, nx0, nx1, wd, invd, lam0, lam1)


def kernel(x, y, lam, bi, edges, senders, receivers):
    n = x.shape[0]
    e = senders.shape[0]
    n_pad = ((n + 1023) // 1024) * 1024
    rows = n_pad // 128

    per_w = e // (NC * NS)
    chunk = 2000
    if per_w % chunk or (per_w // chunk) % 2 or chunk % LANES:
        chunk = LANES
        for cand in range(16, min(per_w, 8192) + 1, 16):
            if per_w % cand == 0 and (per_w // cand) % 2 == 0:
                chunk = cand

    w = edges.reshape(e)

    def col(a, i):
        return a[:, i]

    def pad2d(a):
        return jnp.pad(a, (0, n_pad - n)).reshape(rows, 128)

    lam0, lam1 = col(lam, 0), col(lam, 1)
    y0, y1 = col(y, 0), col(y, 1)

    parts = _edge_pass1(n_pad, e, chunk, lam0, lam1, y0, y1,
                        senders, receivers, w)
    parts2d = tuple(p.reshape(NC, rows, 128) for p in parts)

    nx0, nx1, wd, invd = _node_x(
        pad2d(col(x, 0)), pad2d(col(x, 1)), pad2d(y0), pad2d(y1),
        pad2d(lam0), pad2d(lam1), pad2d(col(bi, 0)), pad2d(col(bi, 1)),
        *parts2d)

    nx0f = nx0.reshape(n_pad)[:n]
    nx1f = nx1.reshape(n_pad)[:n]

    p_xa0, p_xa1 = _edge_pass2(n_pad, e, chunk, nx0f, nx1f,
                               senders, receivers, w)

    ny0, ny1, nl0, nl1 = _node_ylam(
        p_xa0.reshape(NC, rows, 128), p_xa1.reshape(NC, rows, 128),
        nx0, nx1, wd, invd, pad2d(lam0), pad2d(lam1))

    def unpad(a):
        return a.reshape(n_pad)[:n]

    new_x = jnp.stack([nx0f, nx1f], axis=1)
    new_y = jnp.stack([unpad(ny0), unpad(ny1)], axis=1)
    new_lam = jnp.stack([unpad(nl0), unpad(nl1)], axis=1)
    return (new_x, new_y, new_lam)


# speedup vs baseline: 1.3460x; 1.3460x over previous
import jax
import jax.numpy as jnp
from jax import lax
from jax.experimental import pallas as pl
from jax.experimental.pallas import tpu as pltpu
from jax.experimental.pallas import tpu_sc as plsc

NC = 2
NS = 16
LANES = 16


def _mesh():
    return plsc.VectorSubcoreMesh(
        core_axis_name="c", subcore_axis_name="s",
        num_cores=NC, num_subcores=NS)


def _fill(ref, n, value):
    v = jnp.full((LANES,), value, jnp.float32)

    def body(k, carry):
        ref[pl.ds(k * LANES, LANES)] = v
        return carry
    lax.fori_loop(0, n // LANES, body, 0)


def _edge_pass1(n_pad, e, chunk, z0, z1, send, recv, w):
    node_chunk = n_pad // NS
    per_w = e // (NC * NS)
    n_chunks = per_w // chunk
    assert n_chunks % 2 == 0 and n_chunks >= 4

    def body(z0_h, z1_h, send_h, recv_h, w_h,
             o_za0, o_za1, o_wd, o_w2, o_dg,
             sh_za0, sh_za1, sh_wd, sh_w2, sh_dg,
             zb, one_v,
             s_v0, r_v0, w_v0, g0_0, g1_0, rs_0, ws_0, w2_0,
             s_v1, r_v1, w_v1, g0_1, g1_1, rs_1, ws_1, w2_1,
             semL0, semL1, semG0, semG1, semS0, semS1):
        c = lax.axis_index("c")
        s = lax.axis_index("s")
        wid = c * NS + s
        nodelo = s * node_chunk

        tables = (sh_za0, sh_za1, sh_wd, sh_w2, sh_dg)
        outs = (o_za0, o_za1, o_wd, o_w2, o_dg)
        gsrc = (z0_h, z1_h)
        sets = (
            dict(s_v=s_v0, r_v=r_v0, w_v=w_v0, g=(g0_0, g1_0),
                 rs=rs_0, ws=ws_0, w2=w2_0, semL=semL0, semG=semG0,
                 semS=semS0),
            dict(s_v=s_v1, r_v=r_v1, w_v=w_v1, g=(g0_1, g1_1),
                 rs=rs_1, ws=ws_1, w2=w2_1, semL=semL1, semG=semG1,
                 semS=semS1),
        )

        _fill(zb, node_chunk, 0.0)
        for t in tables:
            pltpu.sync_copy(zb, t.at[pl.ds(nodelo, node_chunk)])
        _fill(one_v, chunk, 1.0)
        plsc.subcore_barrier()

        ebase = wid * per_w

        def lin_start(j, st):
            base = ebase + j * chunk
            pltpu.async_copy(send_h.at[pl.ds(base, chunk)], st["s_v"], st["semL"])
            pltpu.async_copy(recv_h.at[pl.ds(base, chunk)], st["r_v"], st["semL"])
            pltpu.async_copy(w_h.at[pl.ds(base, chunk)], st["w_v"], st["semL"])

        def lin_wait(st):
            pltpu.make_async_copy(send_h.at[pl.ds(ebase, chunk)], st["s_v"], st["semL"]).wait()
            pltpu.make_async_copy(recv_h.at[pl.ds(ebase, chunk)], st["r_v"], st["semL"]).wait()
            pltpu.make_async_copy(w_h.at[pl.ds(ebase, chunk)], st["w_v"], st["semL"]).wait()

        def gath_start(st):
            for src, dst in zip(gsrc, st["g"]):
                pltpu.async_copy(src.at[st["s_v"]], dst, st["semG"])

        def gath_wait(st):
            for src, dst in zip(gsrc, st["g"]):
                pltpu.make_async_copy(src.at[st["s_v"]], dst, st["semG"]).wait()

        def compute(st):
            w_v, r_v = st["w_v"], st["r_v"]
            g0, g1 = st["g"]
            rs, ws, w2 = st["rs"], st["ws"], st["w2"]

            def mul_body(k, carry2):
                sl = pl.ds(k * LANES, LANES)
                wv = w_v[sl]
                nw = -wv
                g0[sl] = nw * g0[sl]
                g1[sl] = nw * g1[sl]
                ws[sl] = wv
                w2[sl] = wv * wv
                rs[sl] = r_v[sl]
                return carry2
            lax.fori_loop(0, chunk // LANES, mul_body, 0)

        def scat_srcs(st):
            return st["g"] + (st["ws"], st["w2"], one_v)

        def scat_start(st):
            rs = st["rs"]
            for src, t in zip(scat_srcs(st), tables):
                pltpu.async_copy(src, t.at[rs], st["semS"], add=True)

        def scat_wait(st):
            rs = st["rs"]
            for src, t in zip(scat_srcs(st), tables):
                pltpu.make_async_copy(src, t.at[rs], st["semS"]).wait()

        def stage(j, p, first, do_n1, do_n2):
            cur, nxt = sets[p], sets[1 - p]
            gath_wait(cur)
            compute(cur)
            scat_start(cur)
            if do_n2:
                lin_start(j + 2, cur)
            if do_n1:
                lin_wait(nxt)
                if not first:
                    scat_wait(nxt)
                gath_start(nxt)

        lin_start(0, sets[0])
        lin_wait(sets[0])
        gath_start(sets[0])
        lin_start(1, sets[1])

        stage(0, 0, True, True, True)
        stage(1, 1, False, True, True)

        def pair_body(j2, carry):
            j = 2 + j2 * 2
            stage(j, 0, False, True, True)
            stage(j + 1, 1, False, True, True)
            return carry
        lax.fori_loop(0, (n_chunks - 4) // 2, pair_body, 0)
        stage(n_chunks - 2, 0, False, True, False)
        stage(n_chunks - 1, 1, False, False, False)
        scat_wait(sets[0])
        scat_wait(sets[1])

        plsc.subcore_barrier()
        for t, o in zip(tables, outs):
            pltpu.sync_copy(t.at[pl.ds(nodelo, node_chunk)],
                            o.at[c, pl.ds(nodelo, node_chunk)])

    part = jax.ShapeDtypeStruct((NC, n_pad), jnp.float32)
    ebuf = ([pltpu.VMEM((chunk,), jnp.int32)] * 2
            + [pltpu.VMEM((chunk,), jnp.float32)] * 3
            + [pltpu.VMEM((chunk,), jnp.int32)]
            + [pltpu.VMEM((chunk,), jnp.float32)] * 2)
    fn = pl.kernel(
        body,
        out_type=(part,) * 5,
        mesh=_mesh(),
        scratch_types=(
            [pltpu.VMEM_SHARED((n_pad,), jnp.float32)] * 5
            + [pltpu.VMEM((node_chunk,), jnp.float32),
               pltpu.VMEM((chunk,), jnp.float32)]
            + ebuf + ebuf
            + [pltpu.SemaphoreType.DMA] * 6),
    )
    return fn(z0, z1, send, recv, w)


def _edge_pass2(n_pad, e, chunk, nx0, nx1, send, recv, w):
    node_chunk = n_pad // NS
    per_w = e // (NC * NS)
    n_chunks = per_w // chunk
    assert n_chunks % 2 == 0 and n_chunks >= 4

    def body(nx0_h, nx1_h, send_h, recv_h, w_h,
             o_xa0, o_xa1,
             sh_xa0, sh_xa1,
             zb,
             s_v0, r_v0, w_v0, g0_0, g1_0, rs_0,
             s_v1, r_v1, w_v1, g0_1, g1_1, rs_1,
             semL0, semL1, semG0, semG1, semS0, semS1):
        c = lax.axis_index("c")
        s = lax.axis_index("s")
        wid = c * NS + s
        nodelo = s * node_chunk

        tables = (sh_xa0, sh_xa1)
        gsrc = (nx0_h, nx1_h)
        sets = (
            dict(s_v=s_v0, r_v=r_v0, w_v=w_v0, g=(g0_0, g1_0),
                 rs=rs_0, semL=semL0, semG=semG0, semS=semS0),
            dict(s_v=s_v1, r_v=r_v1, w_v=w_v1, g=(g0_1, g1_1),
                 rs=rs_1, semL=semL1, semG=semG1, semS=semS1),
        )

        _fill(zb, node_chunk, 0.0)
        pltpu.sync_copy(zb, sh_xa0.at[pl.ds(nodelo, node_chunk)])
        pltpu.sync_copy(zb, sh_xa1.at[pl.ds(nodelo, node_chunk)])
        plsc.subcore_barrier()

        ebase = wid * per_w

        def lin_start(j, st):
            base = ebase + j * chunk
            pltpu.async_copy(send_h.at[pl.ds(base, chunk)], st["s_v"], st["semL"])
            pltpu.async_copy(recv_h.at[pl.ds(base, chunk)], st["r_v"], st["semL"])
            pltpu.async_copy(w_h.at[pl.ds(base, chunk)], st["w_v"], st["semL"])

        def lin_wait(st):
            pltpu.make_async_copy(send_h.at[pl.ds(ebase, chunk)], st["s_v"], st["semL"]).wait()
            pltpu.make_async_copy(recv_h.at[pl.ds(ebase, chunk)], st["r_v"], st["semL"]).wait()
            pltpu.make_async_copy(w_h.at[pl.ds(ebase, chunk)], st["w_v"], st["semL"]).wait()

        def gath_start(st):
            for src, dst in zip(gsrc, st["g"]):
                pltpu.async_copy(src.at[st["s_v"]], dst, st["semG"])

        def gath_wait(st):
            for src, dst in zip(gsrc, st["g"]):
                pltpu.make_async_copy(src.at[st["s_v"]], dst, st["semG"]).wait()

        def compute(st):
            w_v, r_v = st["w_v"], st["r_v"]
            g0, g1 = st["g"]
            rs = st["rs"]

            def mul_body(k, carry2):
                sl = pl.ds(k * LANES, LANES)
                nw = -w_v[sl]
                g0[sl] = nw * g0[sl]
                g1[sl] = nw * g1[sl]
                rs[sl] = r_v[sl]
                return carry2
            lax.fori_loop(0, chunk // LANES, mul_body, 0)

        def scat_start(st):
            for src, t in zip(st["g"], tables):
                pltpu.async_copy(src, t.at[st["rs"]], st["semS"], add=True)

        def scat_wait(st):
            for src, t in zip(st["g"], tables):
                pltpu.make_async_copy(src, t.at[st["rs"]], st["semS"]).wait()

        def stage(j, p, first, do_n1, do_n2):
            cur, nxt = sets[p], sets[1 - p]
            gath_wait(cur)
            compute(cur)
            scat_start(cur)
            if do_n2:
                lin_start(j + 2, cur)
            if do_n1:
                lin_wait(nxt)
                if not first:
                    scat_wait(nxt)
                gath_start(nxt)

        lin_start(0, sets[0])
        lin_wait(sets[0])
        gath_start(sets[0])
        lin_start(1, sets[1])

        stage(0, 0, True, True, True)
        stage(1, 1, False, True, True)

        def pair_body(j2, carry):
            j = 2 + j2 * 2
            stage(j, 0, False, True, True)
            stage(j + 1, 1, False, True, True)
            return carry
        lax.fori_loop(0, (n_chunks - 4) // 2, pair_body, 0)
        stage(n_chunks - 2, 0, False, True, False)
        stage(n_chunks - 1, 1, False, False, False)
        scat_wait(sets[0])
        scat_wait(sets[1])

        plsc.subcore_barrier()
        pltpu.sync_copy(sh_xa0.at[pl.ds(nodelo, node_chunk)],
                        o_xa0.at[c, pl.ds(nodelo, node_chunk)])
        pltpu.sync_copy(sh_xa1.at[pl.ds(nodelo, node_chunk)],
                        o_xa1.at[c, pl.ds(nodelo, node_chunk)])

    part = jax.ShapeDtypeStruct((NC, n_pad), jnp.float32)
    ebuf = ([pltpu.VMEM((chunk,), jnp.int32)] * 2
            + [pltpu.VMEM((chunk,), jnp.float32)] * 3
            + [pltpu.VMEM((chunk,), jnp.int32)])
    fn = pl.kernel(
        body,
        out_type=(part, part),
        mesh=_mesh(),
        scratch_types=(
            [pltpu.VMEM_SHARED((n_pad,), jnp.float32)] * 2
            + [pltpu.VMEM((node_chunk,), jnp.float32)]
            + ebuf + ebuf
            + [pltpu.SemaphoreType.DMA] * 6),
    )
    return fn(nx0, nx1, send, recv, w)


def _node_z(lam0, lam1, y0, y1):
    def body(lam0_r, lam1_r, y0_r, y1_r, z0_o, z1_o):
        z0_o[...] = lam0_r[...] + y0_r[...]
        z1_o[...] = lam1_r[...] + y1_r[...]

    out = jax.ShapeDtypeStruct(lam0.shape, jnp.float32)
    return pl.pallas_call(body, out_shape=(out, out))(lam0, lam1, y0, y1)


def _node_x(x0, x1, z0, z1, b0, b1,
            p_za0, p_za1, p_wd, p_w2, p_dg):
    def body(x0_r, x1_r, z0_r, z1_r, b0_r, b1_r,
             za0_r, za1_r, wd_r, w2_r, dg_r,
             nx0_o, nx1_o, wd_o, invd_o):
        za0 = za0_r[0] + za0_r[1]
        za1 = za1_r[0] + za1_r[1]
        wd = wd_r[0] + wd_r[1]
        dsq = w2_r[0] + w2_r[1]
        dg = dg_r[0] + dg_r[1]
        mii = wd * wd + dsq
        den = 1.0 / (2.0 + mii)
        nx0_o[...] = (2.0 * b0_r[...] - za0 - wd * z0_r[...]
                      + mii * x0_r[...]) * den
        nx1_o[...] = (2.0 * b1_r[...] - za1 - wd * z1_r[...]
                      + mii * x1_r[...]) * den
        wd_o[...] = wd
        invd_o[...] = 1.0 / (1.0 + dg)

    shp = x0.shape
    out = jax.ShapeDtypeStruct(shp, jnp.float32)
    return pl.pallas_call(
        body, out_shape=(out,) * 4,
    )(x0, x1, z0, z1, b0, b1,
      p_za0, p_za1, p_wd, p_w2, p_dg)


def _node_ylam(p_xa0, p_xa1, nx0, nx1, wd, invd, lam0, lam1):
    def body(xa0_r, xa1_r, nx0_r, nx1_r, wd_r, invd_r, lam0_r, lam1_r,
             ny0_o, ny1_o, nl0_o, nl1_o):
        xa0 = xa0_r[0] + xa0_r[1]
        xa1 = xa1_r[0] + xa1_r[1]
        ny0 = invd_r[...] * (xa0 + wd_r[...] * nx0_r[...])
        ny1 = invd_r[...] * (xa1 + wd_r[...] * nx1_r[...])
        ny0_o[...] = ny0
        ny1_o[...] = ny1
        nl0_o[...] = lam0_r[...] + ny0
        nl1_o[...] = lam1_r[...] + ny1

    shp = nx0.shape
    out = jax.ShapeDtypeStruct(shp, jnp.float32)
    return pl.pallas_call(
        body, out_shape=(out,) * 4,
    )(p_xa0, p_xa1, nx0, nx1, wd, invd, lam0, lam1)


def kernel(x, y, lam, bi, edges, senders, receivers):
    n = x.shape[0]
    e = senders.shape[0]
    n_pad = ((n + 1023) // 1024) * 1024
    rows = n_pad // 128

    per_w = e // (NC * NS)
    chunk = 2000
    if per_w % chunk or (per_w // chunk) % 2 or chunk % LANES:
        chunk = LANES
        for cand in range(16, min(per_w, 8192) + 1, 16):
            if per_w % cand == 0 and (per_w // cand) % 2 == 0:
                chunk = cand

    w = edges.reshape(e)

    def col(a, i):
        return a[:, i]

    def pad2d(a):
        return jnp.pad(a, (0, n_pad - n)).reshape(rows, 128)

    lam0, lam1 = col(lam, 0), col(lam, 1)
    y0, y1 = col(y, 0), col(y, 1)

    z0p, z1p = _node_z(pad2d(lam0), pad2d(lam1), pad2d(y0), pad2d(y1))
    z0f = z0p.reshape(n_pad)
    z1f = z1p.reshape(n_pad)

    parts = _edge_pass1(n_pad, e, chunk, z0f, z1f,
                        senders, receivers, w)
    parts2d = tuple(p.reshape(NC, rows, 128) for p in parts)

    nx0, nx1, wd, invd = _node_x(
        pad2d(col(x, 0)), pad2d(col(x, 1)), z0p, z1p,
        pad2d(col(bi, 0)), pad2d(col(bi, 1)),
        *parts2d)

    nx0f = nx0.reshape(n_pad)[:n]
    nx1f = nx1.reshape(n_pad)[:n]

    p_xa0, p_xa1 = _edge_pass2(n_pad, e, chunk, nx0f, nx1f,
                               senders, receivers, w)

    ny0, ny1, nl0, nl1 = _node_ylam(
        p_xa0.reshape(NC, rows, 128), p_xa1.reshape(NC, rows, 128),
        nx0, nx1, wd, invd, pad2d(lam0), pad2d(lam1))

    def unpad(a):
        return a.reshape(n_pad)[:n]

    new_x = jnp.stack([nx0f, nx1f], axis=1)
    new_y = jnp.stack([unpad(ny0), unpad(ny1)], axis=1)
    new_lam = jnp.stack([unpad(nl0), unpad(nl1)], axis=1)
    return (new_x, new_y, new_lam)
